# baseline (speedup 1.0000x reference)
import functools

import jax
import jax.numpy as jnp
from jax import lax
from jax.experimental import pallas as pl
from jax.experimental.pallas import tpu as pltpu
from jax.experimental.pallas import tpu_sc as plsc

C_IN = 65
K1 = 128
H1, H2, H3, C_OUT = 1024, 512, 256, 22
TOPK = 8
T_TILE = 2048
LANES = 16


def _mlp_body(x_ref, w1_ref, w2_ref, b2_ref, w3_ref, b3_ref,
              w4_ref, b4_ref, z_ref):
    def dot(a, b):
        return jnp.dot(a, b, preferred_element_type=jnp.float32)

    x = x_ref[0]
    pad = jnp.zeros((K1 - C_IN - 1, T_TILE), jnp.float32)
    ones = jnp.ones((1, T_TILE), jnp.float32)
    xp = jnp.concatenate([x, ones, pad], axis=0)
    h = jnp.maximum(dot(w1_ref[...], xp), 0.0)
    h = jnp.maximum(dot(w2_ref[...], h) + b2_ref[...], 0.0)
    h = jnp.maximum(dot(w3_ref[...], h) + b3_ref[...], 0.0)
    z_ref[0] = dot(w4_ref[...], h) + b4_ref[...]


def _mlp_logits(input, W1, b1, W2, b2, W3, b3, W4, b4):
    B, C, T = input.shape
    nt = T // T_TILE
    grid = (B, nt)
    return pl.pallas_call(
        _mlp_body,
        grid=grid,
        in_specs=[
            pl.BlockSpec((1, C_IN, T_TILE), lambda b, t: (b, 0, t)),
            pl.BlockSpec((H1, K1), lambda b, t: (0, 0)),
            pl.BlockSpec((H2, H1), lambda b, t: (0, 0)),
            pl.BlockSpec((H2, 1), lambda b, t: (0, 0)),
            pl.BlockSpec((H3, H2), lambda b, t: (0, 0)),
            pl.BlockSpec((H3, 1), lambda b, t: (0, 0)),
            pl.BlockSpec((C_OUT, H3), lambda b, t: (0, 0)),
            pl.BlockSpec((C_OUT, 1), lambda b, t: (0, 0)),
        ],
        out_specs=pl.BlockSpec((1, C_OUT, T_TILE), lambda b, t: (b, 0, t)),
        out_shape=jax.ShapeDtypeStruct((B, C_OUT, T), jnp.float32),
    )(
        input,
        jnp.concatenate(
            [W1.T, b1.reshape(H1, 1), jnp.zeros((H1, K1 - C_IN - 1),
                                                jnp.float32)], axis=1),
        W2.T, b2.reshape(H2, 1),
        W3.T, b3.reshape(H3, 1),
        W4.T, b4.reshape(C_OUT, 1),
    )


def _topk_slab(zb):
    ngroups = T_TILE // LANES

    def group(g, _):
        sl = pl.ds(g * LANES, LANES)
        zs = [zb[j, sl] for j in range(C_OUT)]
        neg = jnp.full((LANES,), -jnp.inf, jnp.float32)
        m = [neg] * TOPK
        for j in range(C_OUT):
            x = zs[j]
            for k in range(TOPK):
                hi = jnp.maximum(m[k], x)
                x = jnp.minimum(m[k], x)
                m[k] = hi
        t = m[TOPK - 1]
        cnt = jnp.zeros((LANES,), jnp.int32)
        for j in range(C_OUT):
            cnt = cnt + jnp.where(zs[j] > t, 1, 0)
        for j in range(C_OUT):
            eq = zs[j] == t
            take = jnp.logical_or(zs[j] > t,
                                  jnp.logical_and(eq, cnt < TOPK))
            zb[j, sl] = jnp.where(take, zs[j], 0.0)
            cnt = cnt + jnp.where(eq, 1, 0)
        return 0

    lax.fori_loop(0, ngroups, group, 0)


def _topk_mask(z):
    B, C, T = z.shape
    info = plsc.get_sparse_core_info()
    nw = info.num_cores * info.num_subcores
    nt = T // T_TILE
    slabs_per_w = (B * nt) // nw
    mesh = plsc.VectorSubcoreMesh(core_axis_name="c", subcore_axis_name="s")

    @functools.partial(
        pl.kernel, mesh=mesh,
        out_type=jax.ShapeDtypeStruct((B, C, T), jnp.float32),
        scratch_types=[pltpu.VMEM((C_OUT, T_TILE), jnp.float32)],
    )
    def k(z_hbm, out_hbm, zb):
        wid = lax.axis_index("s") * info.num_cores + lax.axis_index("c")
        for i in range(slabs_per_w):
            s = wid * slabs_per_w + i
            b = s // nt
            t0 = (s % nt) * T_TILE
            pltpu.sync_copy(z_hbm.at[b, :, pl.ds(t0, T_TILE)], zb)
            _topk_slab(zb)
            pltpu.sync_copy(zb, out_hbm.at[b, :, pl.ds(t0, T_TILE)])

    return k(z)


@jax.jit
def kernel(input, W1, b1, W2, b2, W3, b3, W4, b4):
    z = _mlp_logits(input, W1, b1, W2, b2, W3, b3, W4, b4)
    return _topk_mask(z)


# speedup vs baseline: 1.0232x; 1.0232x over previous
import functools

import jax
import jax.numpy as jnp
from jax import lax
from jax.experimental import pallas as pl
from jax.experimental.pallas import tpu as pltpu
from jax.experimental.pallas import tpu_sc as plsc

C_IN = 65
K1 = 128
H1, H2, H3, C_OUT = 1024, 512, 256, 22
TOPK = 8
T_TILE = 2048
LANES = 16


def _mlp_body(x_ref, w1_ref, w2_ref, b2_ref, w3_ref, b3_ref,
              w4_ref, b4_ref, z_ref):
    def dot(a, b):
        return jnp.dot(a, b, preferred_element_type=jnp.float32)

    x = x_ref[0]
    pad = jnp.zeros((K1 - C_IN - 1, T_TILE), jnp.float32)
    ones = jnp.ones((1, T_TILE), jnp.float32)
    xp = jnp.concatenate([x, ones, pad], axis=0)
    h = jnp.maximum(dot(w1_ref[...], xp), 0.0)
    h = jnp.maximum(dot(w2_ref[...], h) + b2_ref[...], 0.0)
    h = jnp.maximum(dot(w3_ref[...], h) + b3_ref[...], 0.0)
    z_ref[0] = dot(w4_ref[...], h) + b4_ref[...]


def _mlp_logits(input, W1, b1, W2, b2, W3, b3, W4, b4, b0, Bc):
    B, C, T = input.shape
    nt = T // T_TILE
    grid = (Bc, nt)
    return pl.pallas_call(
        _mlp_body,
        grid=grid,
        in_specs=[
            pl.BlockSpec((1, C_IN, T_TILE), lambda b, t: (b + b0, 0, t)),
            pl.BlockSpec((H1, K1), lambda b, t: (0, 0)),
            pl.BlockSpec((H2, H1), lambda b, t: (0, 0)),
            pl.BlockSpec((H2, 1), lambda b, t: (0, 0)),
            pl.BlockSpec((H3, H2), lambda b, t: (0, 0)),
            pl.BlockSpec((H3, 1), lambda b, t: (0, 0)),
            pl.BlockSpec((C_OUT, H3), lambda b, t: (0, 0)),
            pl.BlockSpec((C_OUT, 1), lambda b, t: (0, 0)),
        ],
        out_specs=pl.BlockSpec((1, C_OUT, T_TILE), lambda b, t: (b, 0, t)),
        out_shape=jax.ShapeDtypeStruct((Bc, C_OUT, T), jnp.float32),
    )(
        input,
        jnp.concatenate(
            [W1.T, b1.reshape(H1, 1), jnp.zeros((H1, K1 - C_IN - 1),
                                                jnp.float32)], axis=1),
        W2.T, b2.reshape(H2, 1),
        W3.T, b3.reshape(H3, 1),
        W4.T, b4.reshape(C_OUT, 1),
    )


def _topk_slab(zb, width):
    ngroups = width // LANES

    def group(g, _):
        sl = pl.ds(g * LANES, LANES)
        zs = [zb[j, sl] for j in range(C_OUT)]
        neg = jnp.full((LANES,), -jnp.inf, jnp.float32)
        m = [neg] * TOPK
        for j in range(C_OUT):
            x = zs[j]
            for k in range(TOPK):
                hi = jnp.maximum(m[k], x)
                x = jnp.minimum(m[k], x)
                m[k] = hi
        t = m[TOPK - 1]
        cnt = jnp.zeros((LANES,), jnp.int32)
        for j in range(C_OUT):
            cnt = cnt + jnp.where(zs[j] > t, 1, 0)
        for j in range(C_OUT):
            eq = zs[j] == t
            take = jnp.logical_or(zs[j] > t,
                                  jnp.logical_and(eq, cnt < TOPK))
            zb[j, sl] = jnp.where(take, zs[j], 0.0)
            cnt = cnt + jnp.where(eq, 1, 0)
        return 0

    lax.fori_loop(0, ngroups, group, 0)


def _topk_mask(z):
    B, C, T = z.shape
    info = plsc.get_sparse_core_info()
    nw = info.num_cores * info.num_subcores
    sw = max(LANES, (B * T) // nw)
    sw = min(sw, T)
    nt = T // sw
    slabs_per_w = (B * nt) // nw
    mesh = plsc.VectorSubcoreMesh(core_axis_name="c", subcore_axis_name="s")

    @functools.partial(
        pl.kernel, mesh=mesh,
        out_type=jax.ShapeDtypeStruct((B, C, T), jnp.float32),
        scratch_types=[pltpu.VMEM((C_OUT, sw), jnp.float32)],
    )
    def k(z_hbm, out_hbm, zb):
        wid = lax.axis_index("s") * info.num_cores + lax.axis_index("c")
        for i in range(slabs_per_w):
            s = wid * slabs_per_w + i
            b = s // nt
            t0 = (s % nt) * sw
            pltpu.sync_copy(z_hbm.at[b, :, pl.ds(t0, sw)], zb)
            _topk_slab(zb, sw)
            pltpu.sync_copy(zb, out_hbm.at[b, :, pl.ds(t0, sw)])

    return k(z)


@jax.jit
def kernel(input, W1, b1, W2, b2, W3, b3, W4, b4):
    B = input.shape[0]
    G = 4
    Bc = B // G
    outs = []
    for g in range(G):
        z = _mlp_logits(input, W1, b1, W2, b2, W3, b3, W4, b4, g * Bc, Bc)
        outs.append(_topk_mask(z))
    return jnp.concatenate(outs, axis=0)
